# baseline (speedup 1.0000x reference)
import functools

import jax
import jax.numpy as jnp
from jax import lax
from jax.experimental import pallas as pl
from jax.experimental.pallas import tpu as pltpu
from jax.experimental.pallas import tpu_sc as plsc

_N_FEATURES = 12
_SUB = 4
_EXP = _N_FEATURES // _SUB
_BATCH = 16384
_CHUNK = 128

_info = plsc.get_sparse_core_info()
_NC, _NS = _info.num_cores, _info.num_subcores
_NW = _NC * _NS
_E_PER_W = _BATCH * _EXP // _NW
_N_CHUNKS = _E_PER_W // _CHUNK

_mesh = plsc.VectorSubcoreMesh(core_axis_name="c", subcore_axis_name="s")


@functools.partial(
    pl.kernel,
    mesh=_mesh,
    out_type=jax.ShapeDtypeStruct((_BATCH * _EXP, _SUB), jnp.float32),
    compiler_params=pltpu.CompilerParams(use_tc_tiling_on_sc=False),
    scratch_types=[
        [pltpu.VMEM((_CHUNK,), jnp.int32) for _ in range(_N_CHUNKS)],
        [pltpu.VMEM((_CHUNK, _SUB), jnp.float32) for _ in range(_N_CHUNKS)],
        pltpu.SemaphoreType.DMA,
    ],
)
def _gather_rows(idx_hbm, table_hbm, out_hbm, idx_bufs, row_bufs, sem):
    wid = lax.axis_index("s") * _NC + lax.axis_index("c")
    for j in range(_N_CHUNKS):
        pltpu.sync_copy(idx_hbm.at[wid * _N_CHUNKS + j], idx_bufs[j])
    copies = [
        pltpu.async_copy(table_hbm.at[idx_bufs[j]], row_bufs[j], sem)
        for j in range(_N_CHUNKS)
    ]
    for j in range(_N_CHUNKS):
        copies[j].wait()
        pltpu.sync_copy(
            row_bufs[j],
            out_hbm.at[pl.ds((wid * _N_CHUNKS + j) * _CHUNK, _CHUNK)],
        )


def kernel(image_inds, prf_params, prf_model_index, labels_table):
    del prf_params, prf_model_index
    idx = image_inds.astype(jnp.int32)
    idx3 = (idx[:, None] * _EXP + jnp.arange(_EXP, dtype=jnp.int32)[None, :])
    idx2d = idx3.reshape(_NW * _N_CHUNKS, _CHUNK)
    table4 = labels_table.reshape(-1, _SUB)
    flat = _gather_rows(idx2d, table4)
    features = flat.reshape(_BATCH, _N_FEATURES)
    feature_inds_defined = jnp.ones((_N_FEATURES,), dtype=bool)
    return (features, feature_inds_defined)


# speedup vs baseline: 9.8802x; 9.8802x over previous
import functools

import jax
import jax.numpy as jnp
from jax import lax
from jax.experimental import pallas as pl
from jax.experimental.pallas import tpu as pltpu
from jax.experimental.pallas import tpu_sc as plsc

_N_FEATURES = 12
_N_IMAGES = 100000
_BATCH = 16384
_CHUNK = 128

_info = plsc.get_sparse_core_info()
_NC, _NS = _info.num_cores, _info.num_subcores
_NW = _NC * _NS
_B_PER_W = _BATCH // _NW
_N_CHUNKS = _B_PER_W // _CHUNK

_mesh = plsc.VectorSubcoreMesh(core_axis_name="c", subcore_axis_name="s")


@functools.partial(
    pl.kernel,
    mesh=_mesh,
    out_type=jax.ShapeDtypeStruct((_N_FEATURES, _BATCH), jnp.float32),
    compiler_params=pltpu.CompilerParams(use_tc_tiling_on_sc=False),
    scratch_types=[
        pltpu.VMEM((_CHUNK,), jnp.int32),
        [pltpu.VMEM((_CHUNK,), jnp.float32) for _ in range(_N_FEATURES)],
        pltpu.SemaphoreType.DMA,
    ],
)
def _gather_cols(idx_hbm, tab_t_hbm, out_hbm, idx_buf, col_bufs, sem):
    wid = lax.axis_index("s") * _NC + lax.axis_index("c")

    def body(g, carry):
        chunk = wid * _N_CHUNKS + g
        pltpu.sync_copy(idx_hbm.at[chunk], idx_buf)
        copies = [
            pltpu.async_copy(tab_t_hbm.at[c].at[idx_buf], col_bufs[c], sem)
            for c in range(_N_FEATURES)
        ]
        for c in range(_N_FEATURES):
            copies[c].wait()
            pltpu.sync_copy(
                col_bufs[c], out_hbm.at[c].at[pl.ds(chunk * _CHUNK, _CHUNK)]
            )
        return carry

    lax.fori_loop(0, _N_CHUNKS, body, 0)


def kernel(image_inds, prf_params, prf_model_index, labels_table):
    del prf_params, prf_model_index
    idx2d = image_inds.astype(jnp.int32).reshape(_NW * _N_CHUNKS, _CHUNK)
    out_t = _gather_cols(idx2d, labels_table.T)
    features = out_t.T
    feature_inds_defined = jnp.ones((_N_FEATURES,), dtype=bool)
    return (features, feature_inds_defined)


# speedup vs baseline: 10.7824x; 1.0913x over previous
import functools

import jax
import jax.numpy as jnp
from jax import lax
from jax.experimental import pallas as pl
from jax.experimental.pallas import tpu as pltpu
from jax.experimental.pallas import tpu_sc as plsc

_N_FEATURES = 12
_N_IMAGES = 100000
_BATCH = 16384
_CHUNK = 128

_info = plsc.get_sparse_core_info()
_NC, _NS = _info.num_cores, _info.num_subcores
_NW = _NC * _NS
_B_PER_W = _BATCH // _NW
_N_CHUNKS = _B_PER_W // _CHUNK

_mesh = plsc.VectorSubcoreMesh(core_axis_name="c", subcore_axis_name="s")


@functools.partial(
    pl.kernel,
    mesh=_mesh,
    out_type=jax.ShapeDtypeStruct((_N_FEATURES, _BATCH), jnp.float32),
    compiler_params=pltpu.CompilerParams(use_tc_tiling_on_sc=False),
    scratch_types=[
        pltpu.VMEM((_CHUNK,), jnp.int32),
        pltpu.VMEM((_CHUNK,), jnp.int32),
        [pltpu.VMEM((_CHUNK,), jnp.float32) for _ in range(_N_FEATURES)],
        [pltpu.VMEM((_CHUNK,), jnp.float32) for _ in range(_N_FEATURES)],
        pltpu.SemaphoreType.DMA,
        pltpu.SemaphoreType.DMA,
    ],
)
def _gather_cols(idx_hbm, tab_t_hbm, out_hbm, idx_a, idx_b, cols_a, cols_b,
                 sem_a, sem_b):
    wid = lax.axis_index("s") * _NC + lax.axis_index("c")

    def fire(chunk, idx_buf, cols, sem):
        pltpu.sync_copy(idx_hbm.at[chunk], idx_buf)
        return [
            pltpu.async_copy(tab_t_hbm.at[c].at[idx_buf], cols[c], sem)
            for c in range(_N_FEATURES)
        ]

    def drain(chunk, cols, copies):
        for c in range(_N_FEATURES):
            copies[c].wait()
            pltpu.sync_copy(
                cols[c], out_hbm.at[c].at[pl.ds(chunk * _CHUNK, _CHUNK)]
            )

    def body(g, carry):
        chunk_a = wid * _N_CHUNKS + 2 * g
        chunk_b = chunk_a + 1
        cp_a = fire(chunk_a, idx_a, cols_a, sem_a)
        cp_b = fire(chunk_b, idx_b, cols_b, sem_b)
        drain(chunk_a, cols_a, cp_a)
        drain(chunk_b, cols_b, cp_b)
        return carry

    lax.fori_loop(0, _N_CHUNKS // 2, body, 0)


def kernel(image_inds, prf_params, prf_model_index, labels_table):
    del prf_params, prf_model_index
    idx2d = image_inds.astype(jnp.int32).reshape(_NW * _N_CHUNKS, _CHUNK)
    out_t = _gather_cols(idx2d, labels_table.T)
    features = out_t.T
    feature_inds_defined = jnp.ones((_N_FEATURES,), dtype=bool)
    return (features, feature_inds_defined)


# speedup vs baseline: 11.3444x; 1.0521x over previous
import functools

import jax
import jax.numpy as jnp
from jax import lax
from jax.experimental import pallas as pl
from jax.experimental.pallas import tpu as pltpu
from jax.experimental.pallas import tpu_sc as plsc

_N_FEATURES = 12
_N_IMAGES = 100000
_BATCH = 16384
_CHUNK = 128

_info = plsc.get_sparse_core_info()
_NC, _NS = _info.num_cores, _info.num_subcores
_NW = _NC * _NS
_B_PER_W = _BATCH // _NW
_N_CHUNKS = _B_PER_W // _CHUNK

_mesh = plsc.VectorSubcoreMesh(core_axis_name="c", subcore_axis_name="s")


@functools.partial(
    pl.kernel,
    mesh=_mesh,
    out_type=jax.ShapeDtypeStruct((_N_FEATURES, _BATCH), jnp.float32),
    compiler_params=pltpu.CompilerParams(use_tc_tiling_on_sc=False),
    scratch_types=[
        pltpu.VMEM((_CHUNK,), jnp.int32),
        pltpu.VMEM((_CHUNK,), jnp.int32),
        [pltpu.VMEM((_CHUNK,), jnp.float32) for _ in range(_N_FEATURES)],
        [pltpu.VMEM((_CHUNK,), jnp.float32) for _ in range(_N_FEATURES)],
        pltpu.SemaphoreType.DMA,
        pltpu.SemaphoreType.DMA,
        pltpu.SemaphoreType.DMA,
    ],
)
def _gather_cols(idx_hbm, tab_t_hbm, out_hbm, idx_a, idx_b, cols_a, cols_b,
                 sem_a, sem_b, sem_w):
    wid = lax.axis_index("s") * _NC + lax.axis_index("c")

    def fire(chunk, idx_buf, cols, sem):
        pltpu.sync_copy(idx_hbm.at[pl.ds(chunk * _CHUNK, _CHUNK)], idx_buf)
        return [
            pltpu.async_copy(tab_t_hbm.at[c].at[idx_buf], cols[c], sem)
            for c in range(_N_FEATURES)
        ]

    def writeback(chunk, cols, copies):
        out = []
        for c in range(_N_FEATURES):
            copies[c].wait()
            out.append(
                pltpu.async_copy(
                    cols[c],
                    out_hbm.at[c].at[pl.ds(chunk * _CHUNK, _CHUNK)],
                    sem_w,
                )
            )
        return out

    def body(g, carry):
        chunk_a = wid * _N_CHUNKS + 2 * g
        chunk_b = chunk_a + 1
        cp_a = fire(chunk_a, idx_a, cols_a, sem_a)
        cp_b = fire(chunk_b, idx_b, cols_b, sem_b)
        wr = writeback(chunk_a, cols_a, cp_a)
        wr += writeback(chunk_b, cols_b, cp_b)
        for w in wr:
            w.wait()
        return carry

    lax.fori_loop(0, _N_CHUNKS // 2, body, 0)


def kernel(image_inds, prf_params, prf_model_index, labels_table):
    del prf_params, prf_model_index
    out_t = _gather_cols(image_inds.astype(jnp.int32), labels_table.T)
    features = out_t.T
    feature_inds_defined = jnp.ones((_N_FEATURES,), dtype=bool)
    return (features, feature_inds_defined)
